# Initial kernel scaffold; baseline (speedup 1.0000x reference)
import jax
import jax.numpy as jnp
from jax.experimental import pallas as pl


def kernel(x, edge_index, edge_attr, batch_mask, nn_W1, nn_b1, nn_W2, nn_b2, conv1_W, conv1_b, conv2_W, conv2_b, lin1_W, lin1_b, lin_mu_W, lin_mu_b):
    raise NotImplementedError("write your pallas kernel here")


# speedup vs baseline: 14.1446x; 14.1446x over previous
import functools

import jax
import jax.numpy as jnp
from jax import lax
from jax.experimental import pallas as pl
from jax.experimental.pallas import tpu as pltpu
from jax.experimental.pallas import tpu_sc as plsc

N = 10000
E = 320000
D = 128
G = 64
LAT = 64
N_PAD = 10240

CHUNK = 128
NCHUNK = E // CHUNK
NC, NS = 2, 16
NW = NC * NS
ROWS_PER_TILE = N_PAD // NS

BE = 16000
BN = 1024


def _edge_mlp_body(eat_ref, w1t_ref, b1_ref, w2_ref, b2_ref, out_ref):
    h = jnp.dot(w1t_ref[...], eat_ref[...], preferred_element_type=jnp.float32)
    h = jnp.maximum(h + b1_ref[...], 0.0)
    out_ref[...] = (jnp.dot(w2_ref[...], h, preferred_element_type=jnp.float32)
                    + b2_ref[...])


def _edge_mlp(edge_attr_t, w1t, b1c, w2row, b2):
    out = pl.pallas_call(
        _edge_mlp_body,
        grid=(E // BE,),
        in_specs=[
            pl.BlockSpec((16, BE), lambda i: (0, i)),
            pl.BlockSpec((16, 16), lambda i: (0, 0)),
            pl.BlockSpec((16, 1), lambda i: (0, 0)),
            pl.BlockSpec((1, 16), lambda i: (0, 0)),
            pl.BlockSpec((1, 1), lambda i: (0, 0)),
        ],
        out_specs=pl.BlockSpec((1, BE), lambda i: (0, i)),
        out_shape=jax.ShapeDtypeStruct((1, E), jnp.float32),
    )(edge_attr_t, w1t, b1c, w2row, b2)
    return out.reshape(E)


def _scale1_body(x_ref, degp_ref, w_ref, y_ref, dinv_ref):
    deg = degp_ref[0, :] + degp_ref[1, :]
    dinv = jnp.where(deg > 0, lax.rsqrt(jnp.maximum(deg, 1e-12)), 0.0)
    xw = jnp.dot(x_ref[...], w_ref[...], preferred_element_type=jnp.float32)
    y_ref[...] = xw * dinv[:, None]
    dinv_ref[...] = dinv


def _scale1(x_p, degp, w):
    return pl.pallas_call(
        _scale1_body,
        grid=(N_PAD // BN,),
        in_specs=[
            pl.BlockSpec((BN, D), lambda i: (i, 0)),
            pl.BlockSpec((2, BN), lambda i: (0, i)),
            pl.BlockSpec((D, D), lambda i: (0, 0)),
        ],
        out_specs=[
            pl.BlockSpec((BN, D), lambda i: (i, 0)),
            pl.BlockSpec((BN,), lambda i: (i,)),
        ],
        out_shape=[
            jax.ShapeDtypeStruct((N_PAD, D), jnp.float32),
            jax.ShapeDtypeStruct((N_PAD,), jnp.float32),
        ],
    )(x_p, degp, w)


def _combine1_body(p_ref, y1_ref, dinv_ref, b_ref, w2_ref, y2_ref):
    dinv = dinv_ref[...]
    h = (p_ref[0] + p_ref[1] + y1_ref[...]) * dinv[:, None] + b_ref[...]
    h = jnp.maximum(h, 0.0)
    hw = jnp.dot(h, w2_ref[...], preferred_element_type=jnp.float32)
    y2_ref[...] = hw * dinv[:, None]


def _combine1(p1, y1, dinv, b, w2):
    return pl.pallas_call(
        _combine1_body,
        grid=(N_PAD // BN,),
        in_specs=[
            pl.BlockSpec((2, BN, D), lambda i: (0, i, 0)),
            pl.BlockSpec((BN, D), lambda i: (i, 0)),
            pl.BlockSpec((BN,), lambda i: (i,)),
            pl.BlockSpec((1, D), lambda i: (0, 0)),
            pl.BlockSpec((D, D), lambda i: (0, 0)),
        ],
        out_specs=pl.BlockSpec((BN, D), lambda i: (i, 0)),
        out_shape=jax.ShapeDtypeStruct((N_PAD, D), jnp.float32),
    )(p1, y1, dinv, b, w2)


def _final_body(p_ref, y2_ref, dinv_ref, b_ref, mask_ref, l1w_ref, l1b_ref,
                mw_ref, mb_ref, out_ref, acc_ref, cnt_ref):
    i = pl.program_id(0)

    @pl.when(i == 0)
    def _():
        acc_ref[...] = jnp.zeros_like(acc_ref)
        cnt_ref[...] = jnp.zeros_like(cnt_ref)

    dinv = dinv_ref[...]
    h = (p_ref[0] + p_ref[1] + y2_ref[...]) * dinv[:, None] + b_ref[...]
    h = jnp.maximum(h, 0.0)
    oh = (lax.broadcasted_iota(jnp.int32, (G, BN), 0)
          == mask_ref[...][None, :]).astype(jnp.float32)
    acc_ref[...] += jnp.dot(oh, h, preferred_element_type=jnp.float32)
    cnt_ref[...] += jnp.sum(oh, axis=1)

    @pl.when(i == pl.num_programs(0) - 1)
    def _():
        pooled = acc_ref[...] / jnp.maximum(cnt_ref[...], 1.0)[:, None]
        emb = jnp.dot(pooled, l1w_ref[...], preferred_element_type=jnp.float32)
        emb = jnp.maximum(emb + l1b_ref[...], 0.0)
        out_ref[...] = (jnp.dot(emb, mw_ref[...],
                                preferred_element_type=jnp.float32)
                        + mb_ref[...])


def _final(p2, y2, dinv, b, mask_p, l1w, l1b, mw, mb):
    return pl.pallas_call(
        _final_body,
        grid=(N_PAD // BN,),
        in_specs=[
            pl.BlockSpec((2, BN, D), lambda i: (0, i, 0)),
            pl.BlockSpec((BN, D), lambda i: (i, 0)),
            pl.BlockSpec((BN,), lambda i: (i,)),
            pl.BlockSpec((1, D), lambda i: (0, 0)),
            pl.BlockSpec((BN,), lambda i: (i,)),
            pl.BlockSpec((D, D), lambda i: (0, 0)),
            pl.BlockSpec((1, D), lambda i: (0, 0)),
            pl.BlockSpec((D, LAT), lambda i: (0, 0)),
            pl.BlockSpec((1, LAT), lambda i: (0, 0)),
        ],
        out_specs=pl.BlockSpec((G, LAT), lambda i: (0, 0)),
        out_shape=jax.ShapeDtypeStruct((G, LAT), jnp.float32),
        scratch_shapes=[
            pltpu.VMEM((G, D), jnp.float32),
            pltpu.VMEM((G,), jnp.float32),
        ],
    )(p2, y2, dinv, b, mask_p, l1w, l1b, mw, mb)


_SC_MESH = plsc.VectorSubcoreMesh(core_axis_name="c", subcore_axis_name="s")


def _deg_body(ew_hbm, col_hbm, onz_hbm, out_hbm, idxc, eww, deg_sp):
    c = lax.axis_index("c")
    s = lax.axis_index("s")
    wid = s * NC + c
    b0 = s * ROWS_PER_TILE
    pltpu.sync_copy(onz_hbm.at[c, pl.ds(b0, ROWS_PER_TILE)],
                    deg_sp.at[pl.ds(b0, ROWS_PER_TILE)])
    plsc.subcore_barrier()
    nmine = (NCHUNK - wid + NW - 1) // NW

    def body(i, carry):
        bb = (wid + i * NW) * CHUNK
        pltpu.sync_copy(col_hbm.at[pl.ds(bb, CHUNK)], idxc)
        pltpu.sync_copy(ew_hbm.at[pl.ds(bb, CHUNK)], eww)
        pltpu.sync_copy(eww, deg_sp.at[idxc], add=True)
        return carry

    lax.fori_loop(0, nmine, body, 0)
    plsc.subcore_barrier()
    pltpu.sync_copy(deg_sp.at[pl.ds(b0, ROWS_PER_TILE)],
                    out_hbm.at[c, pl.ds(b0, ROWS_PER_TILE)])


_deg_kernel = pl.kernel(
    _deg_body,
    out_type=jax.ShapeDtypeStruct((NC, N_PAD), jnp.float32),
    mesh=_SC_MESH,
    scratch_types=[
        pltpu.VMEM((CHUNK,), jnp.int32),
        pltpu.VMEM((CHUNK,), jnp.float32),
        pltpu.VMEM_SHARED((N_PAD,), jnp.float32),
    ],
)


def _msg_body(y_hbm, row_hbm, col_hbm, ew_hbm, z_hbm, out_hbm,
              idxr, idxc, eww, rows, acc_sp, semg):
    c = lax.axis_index("c")
    s = lax.axis_index("s")
    wid = s * NC + c
    b0 = s * ROWS_PER_TILE
    pltpu.sync_copy(z_hbm.at[pl.ds(b0, ROWS_PER_TILE)],
                    acc_sp.at[pl.ds(b0, ROWS_PER_TILE)])
    plsc.subcore_barrier()
    nmine = (NCHUNK - wid + NW - 1) // NW

    def body(i, carry):
        bb = (wid + i * NW) * CHUNK
        pltpu.sync_copy(row_hbm.at[pl.ds(bb, CHUNK)], idxr)
        gath = pltpu.async_copy(y_hbm.at[idxr], rows, semg)
        pltpu.sync_copy(ew_hbm.at[pl.ds(bb, CHUNK)], eww)
        pltpu.sync_copy(col_hbm.at[pl.ds(bb, CHUNK)], idxc)
        gath.wait()

        def scale(g, cc):
            evec = eww[pl.ds(g * 16, 16)]
            for e16 in range(16):
                sv = evec[e16]
                r = g * 16 + e16
                for t in range(8):
                    rows[r, pl.ds(t * 16, 16)] = rows[r, pl.ds(t * 16, 16)] * sv
            return cc

        lax.fori_loop(0, CHUNK // 16, scale, 0)
        pltpu.sync_copy(rows, acc_sp.at[idxc], add=True)
        return carry

    lax.fori_loop(0, nmine, body, 0)
    plsc.subcore_barrier()
    pltpu.sync_copy(acc_sp.at[pl.ds(b0, ROWS_PER_TILE)],
                    out_hbm.at[c, pl.ds(b0, ROWS_PER_TILE)])


_msg_kernel = pl.kernel(
    _msg_body,
    out_type=jax.ShapeDtypeStruct((NC, N_PAD, D), jnp.float32),
    mesh=_SC_MESH,
    scratch_types=[
        pltpu.VMEM((CHUNK,), jnp.int32),
        pltpu.VMEM((CHUNK,), jnp.int32),
        pltpu.VMEM((CHUNK,), jnp.float32),
        pltpu.VMEM((CHUNK, D), jnp.float32),
        pltpu.VMEM_SHARED((N_PAD, D), jnp.float32),
        pltpu.SemaphoreType.DMA,
    ],
)


def kernel(x, edge_index, edge_attr, batch_mask, nn_W1, nn_b1, nn_W2, nn_b2,
           conv1_W, conv1_b, conv2_W, conv2_b, lin1_W, lin1_b,
           lin_mu_W, lin_mu_b):
    row = edge_index[0]
    col = edge_index[1]
    x_p = jnp.pad(x, ((0, N_PAD - N), (0, 0)))
    mask_p = jnp.pad(batch_mask, (0, N_PAD - N), constant_values=G)
    onz = jnp.stack([jnp.ones((N_PAD,), jnp.float32),
                     jnp.zeros((N_PAD,), jnp.float32)])
    zeros2d = jnp.zeros((N_PAD, D), jnp.float32)

    ew = _edge_mlp(edge_attr.T, nn_W1.T, nn_b1.reshape(16, 1),
                   nn_W2.reshape(1, 16), nn_b2.reshape(1, 1))
    degp = _deg_kernel(ew, col, onz)
    y1, dinv = _scale1(x_p, degp, conv1_W)
    p1 = _msg_kernel(y1, row, col, ew, zeros2d)
    y2 = _combine1(p1, y1, dinv, conv1_b.reshape(1, D), conv2_W)
    p2 = _msg_kernel(y2, row, col, ew, zeros2d)
    mu = _final(p2, y2, dinv, conv2_b.reshape(1, D), mask_p,
                lin1_W, lin1_b.reshape(1, D), lin_mu_W, lin_mu_b.reshape(1, LAT))
    return mu
